# Initial kernel scaffold; baseline (speedup 1.0000x reference)
import jax
import jax.numpy as jnp
from jax.experimental import pallas as pl


def kernel(items, W):
    raise NotImplementedError("write your pallas kernel here")


# speedup vs baseline: 1.9108x; 1.9108x over previous
import functools

import jax
import jax.numpy as jnp
from jax import lax
from jax.experimental import pallas as pl
from jax.experimental.pallas import tpu as pltpu
from jax.experimental.pallas import tpu_sc as plsc

_N = 1048576
_D = 64
_CODE_LEN = 20
_NB = 1048576

_BR = 8192


def _bucket_body(x_ref, w_ref, out_ref):
    y = jnp.dot(x_ref[...], w_ref[...], preferred_element_type=jnp.float32)
    k = lax.broadcasted_iota(jnp.int32, y.shape, 1)
    vals = jnp.where(y > 0, jnp.int32(1) << (_CODE_LEN - 1 - k), 0)
    out_ref[...] = jnp.sum(vals, axis=1)


_bucket_call = pl.pallas_call(
    _bucket_body,
    grid=(_N // _BR,),
    in_specs=[
        pl.BlockSpec((_BR, _D), lambda i: (i, 0)),
        pl.BlockSpec((_D, _CODE_LEN), lambda i: (0, 0)),
    ],
    out_specs=pl.BlockSpec((_BR,), lambda i: (i,)),
    out_shape=jax.ShapeDtypeStruct((_N,), jnp.int32),
)

_NC = 2
_NS = 16
_NW = _NC * _NS
_ROWS = _N // 128
_RPT = _ROWS // _NW

_sc_mesh = plsc.VectorSubcoreMesh(core_axis_name="c", subcore_axis_name="s")


@functools.partial(
    pl.kernel,
    out_type=jax.ShapeDtypeStruct((_NC, _NB), jnp.float32),
    mesh=_sc_mesh,
    scratch_types=[
        pltpu.VMEM((_RPT, 128), jnp.int32),
        pltpu.VMEM((128,), jnp.float32),
        pltpu.VMEM_SHARED((_NB,), jnp.float32),
    ],
)
def _hist_call(buckets_hbm, zeros_hbm, out_hbm, idx_v, ones_v, hist_sh):
    c = lax.axis_index("c")
    s = lax.axis_index("s")
    wid = c * _NS + s
    for i in range(8):
        ones_v[pl.ds(i * 16, 16)] = jnp.ones((16,), jnp.float32)
    @pl.when(s == 0)
    def _():
        pltpu.sync_copy(zeros_hbm, hist_sh)

    pltpu.sync_copy(buckets_hbm.at[pl.ds(wid * _RPT, _RPT)], idx_v)
    plsc.subcore_barrier()

    def body(j, carry):
        pltpu.sync_copy(ones_v, hist_sh.at[idx_v.at[j]], add=True)
        return carry

    lax.fori_loop(0, _RPT, body, 0)
    plsc.subcore_barrier()

    @pl.when(s == 0)
    def _():
        pltpu.sync_copy(hist_sh, out_hbm.at[c])


def _merge_body(p_ref, o_ref):
    o_ref[...] = p_ref[0, :] + p_ref[1, :]


_merge_call = pl.pallas_call(
    _merge_body,
    out_shape=jax.ShapeDtypeStruct((_NB,), jnp.float32),
)


def kernel(items, W):
    buckets = _bucket_call(items, W)
    zeros = jnp.zeros((_NB,), jnp.float32)
    partials = _hist_call(buckets.reshape(_ROWS, 128), zeros)
    counts = _merge_call(partials)
    return buckets, counts
